# baseline (speedup 1.0000x reference)
import functools

import jax
import jax.numpy as jnp
from jax import lax
from jax.experimental import pallas as pl
from jax.experimental.pallas import tpu as pltpu
from jax.experimental.pallas import tpu_sc as plsc

B = 16384
D = 640
H = 256
E = 8

M = 256
P = B + E * M
T = P // M

NC = 2
NS = 16
NW = NC * NS

NSEG = D // 128
CHUNK_A = 96
CHUNK_C = 128
RPW_A = P // NW
RPW_C = (2 * B) // NW

MB = 512

_S = 512
_NB = B // _S


def _side_indices(bk):
    oh = (bk[:, None] == jnp.arange(E, dtype=jnp.int32)[None, :]).astype(
        jnp.float32)
    oh3 = oh.reshape(_NB, _S, E)
    tri = jnp.tril(jnp.ones((_S, _S), jnp.float32))
    local = jnp.einsum('ij,bje->bie', tri, oh3)
    bsum = oh3.sum(axis=1)
    boff = jnp.cumsum(bsum, axis=0) - bsum
    counts = bsum.sum(axis=0)
    padded = jnp.ceil(counts / M) * M
    ends = jnp.cumsum(padded)
    starts = ends - padded
    rank = (oh3 * (local + boff[:, None, :])).sum(-1).reshape(B) - 1.0
    pos = (oh @ starts + rank).astype(jnp.int32)
    gidx = jnp.zeros((P,), jnp.int32).at[pos].set(
        jnp.arange(B, dtype=jnp.int32))
    tile_start = jnp.arange(T, dtype=jnp.float32)[:, None] * M
    eot = jnp.minimum(
        (tile_start >= ends[None, :]).sum(axis=1), E - 1
    ).astype(jnp.int32)
    return gidx, pos, eot


def _run_pipeline(gd, sd):
    n = len(gd)
    gd[0].start()
    for j in range(n):
        if j + 1 < n:
            if j >= 1:
                sd[j - 1].wait()
            gd[j + 1].start()
        gd[j].wait()
        sd[j].start()
    if n >= 2:
        sd[n - 2].wait()
    sd[n - 1].wait()


def _pipelined_gather(table, idx_all, out_hbm, base, nch, chunk, bufs,
                      gsems, ssems):
    gd, sd = [], []
    for j in range(nch):
        b = j % 2
        gd.append(pltpu.make_async_copy(
            table.at[idx_all.at[pl.ds(j * chunk, chunk)]], bufs[b], gsems[b]))
        sd.append(pltpu.make_async_copy(
            bufs[b], out_hbm.at[pl.ds(base + j * chunk, chunk)], ssems[b]))
    _run_pipeline(gd, sd)


def _sc_gather_features(wseg, bseg, gidx5):
    mesh = plsc.VectorSubcoreMesh(core_axis_name="c", subcore_axis_name="s")
    nch = RPW_A // CHUNK_A

    @functools.partial(
        pl.kernel,
        out_type=jax.ShapeDtypeStruct((NSEG * 2 * P, 128), jnp.float32),
        mesh=mesh,
        scratch_types=[
            pltpu.VMEM((RPW_A,), jnp.int32),
            pltpu.VMEM((CHUNK_A, 128), jnp.float32),
            pltpu.VMEM((CHUNK_A, 128), jnp.float32),
            pltpu.SemaphoreType.DMA,
            pltpu.SemaphoreType.DMA,
            pltpu.SemaphoreType.DMA,
            pltpu.SemaphoreType.DMA,
        ],
    )
    def k(wseg_hbm, bseg_hbm, gidx5_hbm, out_hbm,
          idx_v, buf0, buf1, g0, g1, s0, s1):
        wid = lax.axis_index("s") * NC + lax.axis_index("c")
        base = wid * RPW_A
        for seg in range(NSEG):
            pltpu.sync_copy(gidx5_hbm.at[pl.ds(seg * 2 * P + base, RPW_A)],
                            idx_v)
            _pipelined_gather(wseg_hbm, idx_v, out_hbm, seg * 2 * P + base,
                              nch, CHUNK_A, (buf0, buf1), (g0, g1), (s0, s1))
        for seg in range(NSEG):
            pltpu.sync_copy(
                gidx5_hbm.at[pl.ds(seg * 2 * P + P + base, RPW_A)], idx_v)
            _pipelined_gather(bseg_hbm, idx_v, out_hbm,
                              seg * 2 * P + P + base, nch, CHUNK_A,
                              (buf0, buf1), (g0, g1), (s0, s1))

    return k(wseg, bseg, gidx5)


def _mm_body(e_ref, x_ref, w_ref, w1_ref, o_ref, acc_ref):
    k = pl.program_id(1)
    part = lax.dot_general(
        x_ref[...].astype(jnp.bfloat16), w_ref[0].astype(jnp.bfloat16),
        (((1,), (1,)), ((), ())),
        preferred_element_type=jnp.float32,
    )

    @pl.when(k == 0)
    def _():
        acc_ref[...] = part

    @pl.when(k != 0)
    def _():
        acc_ref[...] += part

    @pl.when(k == NSEG - 1)
    def _():
        h = jnp.maximum(acc_ref[...], 0.0)
        o_ref[...] = lax.dot_general(
            h.astype(jnp.bfloat16), w1_ref[0].astype(jnp.bfloat16),
            (((1,), (1,)), ((), ())),
            preferred_element_type=jnp.float32,
        )


def _tc_expert_matmul(eot_all, xs, W_ft, W1s):
    return pl.pallas_call(
        _mm_body,
        grid_spec=pltpu.PrefetchScalarGridSpec(
            num_scalar_prefetch=1,
            grid=(2 * T, NSEG),
            in_specs=[
                pl.BlockSpec((M, 128), lambda i, k, e: (k * 2 * T + i, 0)),
                pl.BlockSpec((1, H, 128), lambda i, k, e: (e[i], 0, k)),
                pl.BlockSpec((1, 128, H), lambda i, k, e: (i // T, 0, 0)),
            ],
            out_specs=pl.BlockSpec((M, 128), lambda i, k, e: (i, 0)),
            scratch_shapes=[pltpu.VMEM((M, H), jnp.float32)],
        ),
        out_shape=jax.ShapeDtypeStruct((2 * P, 128), jnp.float32),
        compiler_params=pltpu.CompilerParams(
            dimension_semantics=("arbitrary", "arbitrary"),
        ),
    )(eot_all, xs, W_ft, W1s)


def _sc_gather_hidden(hid, pos_all):
    mesh = plsc.VectorSubcoreMesh(core_axis_name="c", subcore_axis_name="s")
    nch = RPW_C // CHUNK_C

    @functools.partial(
        pl.kernel,
        out_type=jax.ShapeDtypeStruct((2 * B, 128), jnp.float32),
        mesh=mesh,
        scratch_types=[
            pltpu.VMEM((RPW_C,), jnp.int32),
            pltpu.VMEM((CHUNK_C, 128), jnp.float32),
            pltpu.VMEM((CHUNK_C, 128), jnp.float32),
            pltpu.SemaphoreType.DMA,
            pltpu.SemaphoreType.DMA,
            pltpu.SemaphoreType.DMA,
            pltpu.SemaphoreType.DMA,
        ],
    )
    def k(hid_hbm, pos_hbm, out_hbm, idx_all, buf0, buf1, g0, g1, s0, s1):
        wid = lax.axis_index("s") * NC + lax.axis_index("c")
        base = wid * RPW_C
        pltpu.sync_copy(pos_hbm.at[pl.ds(base, RPW_C)], idx_all)
        _pipelined_gather(hid_hbm, idx_all, out_hbm, base, nch, CHUNK_C,
                          (buf0, buf1), (g0, g1), (s0, s1))

    return k(hid, pos_all)


def _mlp_body(yw, yb, b1, w2, b2, w3, b3, stm, o):
    cd = (((1,), (1,)), ((), ()))
    h = jnp.maximum(yw[...] + yb[...] + b1[...], 0.0)
    h = jnp.maximum(
        lax.dot_general(h, w2[...], cd, preferred_element_type=jnp.float32) + b2[...],
        0.0)
    out = jnp.sum(h * w3[...], axis=1, keepdims=True) + b3[0, 0]
    o[...] = out * stm[...]


def _tc_mlp(aligned, b1, W2, b2, W3, b3, stm2):
    nb = B // MB
    return pl.pallas_call(
        _mlp_body,
        grid=(nb,),
        in_specs=[
            pl.BlockSpec((MB, 128), lambda i: (i, 0)),
            pl.BlockSpec((MB, 128), lambda i: (i + nb, 0)),
            pl.BlockSpec((1, 128), lambda i: (0, 0)),
            pl.BlockSpec((32, 128), lambda i: (0, 0)),
            pl.BlockSpec((1, 32), lambda i: (0, 0)),
            pl.BlockSpec((1, 32), lambda i: (0, 0)),
            pl.BlockSpec((1, 1), lambda i: (0, 0)),
            pl.BlockSpec((MB, 1), lambda i: (i, 0)),
        ],
        out_specs=pl.BlockSpec((MB, 1), lambda i: (i, 0)),
        out_shape=jax.ShapeDtypeStruct((B, 1), jnp.float32),
        compiler_params=pltpu.CompilerParams(
            dimension_semantics=("arbitrary",),
        ),
    )(aligned, aligned, b1, W2, b2, W3, b3, stm2)


def kernel(white_features, black_features, white_bucket, black_bucket, stm,
           W_ft, W1, b1, W2, b2, W3, b3):
    wb = (white_bucket % E).astype(jnp.int32)
    bb = (black_bucket % E).astype(jnp.int32)

    gw, posw, ew = _side_indices(wb)
    gb, posb, eb = _side_indices(bb)
    gidx_all = jnp.concatenate([gw, gb])
    eot_all = jnp.concatenate([ew, eb])
    pos_all = jnp.concatenate([posw, P + posb])
    gidx5 = (gidx_all[None, :] * NSEG
             + jnp.arange(NSEG, dtype=jnp.int32)[:, None]).reshape(-1)

    W1s = jnp.zeros((2, 128, H), jnp.float32)
    W1s = W1s.at[0, :32].set(W1[:, :H]).at[1, :32].set(W1[:, H:])
    b1p = jnp.zeros((1, 128), jnp.float32).at[0, :32].set(b1)
    W2p = jnp.zeros((32, 128), jnp.float32).at[:, :32].set(W2)

    wseg = white_features.reshape(NSEG * B, 128)
    bseg = black_features.reshape(NSEG * B, 128)

    xs = _sc_gather_features(wseg, bseg, gidx5)
    y = _tc_expert_matmul(eot_all, xs, W_ft, W1s)
    aligned = _sc_gather_hidden(y, pos_all)
    out = _tc_mlp(
        aligned,
        b1p, W2p, b2.reshape(1, 32),
        W3, b3.reshape(1, 1),
        stm.reshape(B, 1),
    )
    return out


# speedup vs baseline: 2.2529x; 2.2529x over previous
import functools

import jax
import jax.numpy as jnp
from jax import lax
from jax.experimental import pallas as pl
from jax.experimental.pallas import tpu as pltpu
from jax.experimental.pallas import tpu_sc as plsc

B = 16384
D = 640
H = 256
E = 8

M = 256
P = B + E * M
T = P // M

NC = 2
NS = 16
NW = NC * NS

CHUNK_A = 48
CHUNK_C = 128
RPW_A = P // NW
RPW_C = (2 * B) // NW

MB = 512

_S = 512
_NB = B // _S


def _side_indices(bk):
    oh = (bk[:, None] == jnp.arange(E, dtype=jnp.int32)[None, :]).astype(
        jnp.float32)
    oh3 = oh.reshape(_NB, _S, E)
    tri = jnp.tril(jnp.ones((_S, _S), jnp.float32))
    local = jnp.einsum('ij,bje->bie', tri, oh3)
    bsum = oh3.sum(axis=1)
    boff = jnp.cumsum(bsum, axis=0) - bsum
    counts = bsum.sum(axis=0)
    padded = jnp.ceil(counts / M) * M
    ends = jnp.cumsum(padded)
    starts = ends - padded
    rank = (oh3 * (local + boff[:, None, :])).sum(-1).reshape(B) - 1.0
    pos = (oh @ starts + rank).astype(jnp.int32)
    gidx = jnp.zeros((P,), jnp.int32).at[pos].set(
        jnp.arange(B, dtype=jnp.int32))
    tile_start = jnp.arange(T, dtype=jnp.float32)[:, None] * M
    eot = jnp.minimum(
        (tile_start >= ends[None, :]).sum(axis=1), E - 1
    ).astype(jnp.int32)
    return gidx, pos, eot


def _run_pipeline(gd, sd):
    n = len(gd)
    gd[0].start()
    for j in range(n):
        if j + 1 < n:
            if j >= 1:
                sd[j - 1].wait()
            gd[j + 1].start()
        gd[j].wait()
        sd[j].start()
    if n >= 2:
        sd[n - 2].wait()
    sd[n - 1].wait()


def _pipelined_gather(table, idx_all, out_hbm, base, nch, chunk, bufs,
                      gsems, ssems):
    gd, sd = [], []
    for j in range(nch):
        b = j % 2
        gd.append(pltpu.make_async_copy(
            table.at[idx_all.at[pl.ds(j * chunk, chunk)]], bufs[b], gsems[b]))
        sd.append(pltpu.make_async_copy(
            bufs[b], out_hbm.at[pl.ds(base + j * chunk, chunk)], ssems[b]))
    _run_pipeline(gd, sd)


def _sc_gather_features(white, black, gidx_all):
    mesh = plsc.VectorSubcoreMesh(core_axis_name="c", subcore_axis_name="s")
    nch = RPW_A // CHUNK_A

    @functools.partial(
        pl.kernel,
        out_type=jax.ShapeDtypeStruct((2 * P, D), jnp.float32),
        mesh=mesh,
        scratch_types=[
            pltpu.VMEM((RPW_A,), jnp.int32),
            pltpu.VMEM((CHUNK_A, D), jnp.float32),
            pltpu.VMEM((CHUNK_A, D), jnp.float32),
            pltpu.SemaphoreType.DMA,
            pltpu.SemaphoreType.DMA,
            pltpu.SemaphoreType.DMA,
            pltpu.SemaphoreType.DMA,
        ],
    )
    def k(white_hbm, black_hbm, gidx_hbm, out_hbm,
          idx_v, buf0, buf1, g0, g1, s0, s1):
        wid = lax.axis_index("s") * NC + lax.axis_index("c")
        base = wid * RPW_A
        pltpu.sync_copy(gidx_hbm.at[pl.ds(base, RPW_A)], idx_v)
        _pipelined_gather(white_hbm, idx_v, out_hbm, base, nch, CHUNK_A,
                          (buf0, buf1), (g0, g1), (s0, s1))
        pltpu.sync_copy(gidx_hbm.at[pl.ds(P + base, RPW_A)], idx_v)
        _pipelined_gather(black_hbm, idx_v, out_hbm, P + base, nch, CHUNK_A,
                          (buf0, buf1), (g0, g1), (s0, s1))

    return k(white, black, gidx_all)


def _mm_body(e_ref, x_ref, w_ref, w1_ref, o_ref):
    acc = lax.dot_general(
        x_ref[...].astype(jnp.bfloat16), w_ref[0].astype(jnp.bfloat16),
        (((1,), (1,)), ((), ())),
        preferred_element_type=jnp.float32,
    )
    h = jnp.maximum(acc, 0.0)
    o_ref[...] = lax.dot_general(
        h.astype(jnp.bfloat16), w1_ref[0].astype(jnp.bfloat16),
        (((1,), (1,)), ((), ())),
        preferred_element_type=jnp.float32,
    )


def _tc_expert_matmul(eot_all, xs, W_ft, W1s):
    return pl.pallas_call(
        _mm_body,
        grid_spec=pltpu.PrefetchScalarGridSpec(
            num_scalar_prefetch=1,
            grid=(2 * T,),
            in_specs=[
                pl.BlockSpec((M, D), lambda i, e: (i, 0)),
                pl.BlockSpec((1, H, D), lambda i, e: (e[i], 0, 0)),
                pl.BlockSpec((1, 128, H), lambda i, e: (i // T, 0, 0)),
            ],
            out_specs=pl.BlockSpec((M, 128), lambda i, e: (i, 0)),
        ),
        out_shape=jax.ShapeDtypeStruct((2 * P, 128), jnp.float32),
        compiler_params=pltpu.CompilerParams(
            dimension_semantics=("arbitrary",),
        ),
    )(eot_all, xs, W_ft, W1s)


def _sc_gather_hidden(hid, pos_all):
    mesh = plsc.VectorSubcoreMesh(core_axis_name="c", subcore_axis_name="s")
    nch = RPW_C // CHUNK_C

    @functools.partial(
        pl.kernel,
        out_type=jax.ShapeDtypeStruct((2 * B, 128), jnp.float32),
        mesh=mesh,
        scratch_types=[
            pltpu.VMEM((RPW_C,), jnp.int32),
            pltpu.VMEM((CHUNK_C, 128), jnp.float32),
            pltpu.VMEM((CHUNK_C, 128), jnp.float32),
            pltpu.SemaphoreType.DMA,
            pltpu.SemaphoreType.DMA,
            pltpu.SemaphoreType.DMA,
            pltpu.SemaphoreType.DMA,
        ],
    )
    def k(hid_hbm, pos_hbm, out_hbm, idx_all, buf0, buf1, g0, g1, s0, s1):
        wid = lax.axis_index("s") * NC + lax.axis_index("c")
        base = wid * RPW_C
        pltpu.sync_copy(pos_hbm.at[pl.ds(base, RPW_C)], idx_all)
        _pipelined_gather(hid_hbm, idx_all, out_hbm, base, nch, CHUNK_C,
                          (buf0, buf1), (g0, g1), (s0, s1))

    return k(hid, pos_all)


def _mlp_body(yw, yb, b1, w2, b2, w3, b3, stm, o):
    cd = (((1,), (1,)), ((), ()))
    h = jnp.maximum(yw[...] + yb[...] + b1[...], 0.0)
    h = jnp.maximum(
        lax.dot_general(h, w2[...], cd, preferred_element_type=jnp.float32) + b2[...],
        0.0)
    out = jnp.sum(h * w3[...], axis=1, keepdims=True) + b3[0, 0]
    o[...] = out * stm[...]


def _tc_mlp(aligned, b1, W2, b2, W3, b3, stm2):
    nb = B // MB
    return pl.pallas_call(
        _mlp_body,
        grid=(nb,),
        in_specs=[
            pl.BlockSpec((MB, 128), lambda i: (i, 0)),
            pl.BlockSpec((MB, 128), lambda i: (i + nb, 0)),
            pl.BlockSpec((1, 128), lambda i: (0, 0)),
            pl.BlockSpec((32, 128), lambda i: (0, 0)),
            pl.BlockSpec((1, 32), lambda i: (0, 0)),
            pl.BlockSpec((1, 32), lambda i: (0, 0)),
            pl.BlockSpec((1, 1), lambda i: (0, 0)),
            pl.BlockSpec((MB, 1), lambda i: (i, 0)),
        ],
        out_specs=pl.BlockSpec((MB, 1), lambda i: (i, 0)),
        out_shape=jax.ShapeDtypeStruct((B, 1), jnp.float32),
        compiler_params=pltpu.CompilerParams(
            dimension_semantics=("arbitrary",),
        ),
    )(aligned, aligned, b1, W2, b2, W3, b3, stm2)


def kernel(white_features, black_features, white_bucket, black_bucket, stm,
           W_ft, W1, b1, W2, b2, W3, b3):
    wb = (white_bucket % E).astype(jnp.int32)
    bb = (black_bucket % E).astype(jnp.int32)

    gw, posw, ew = _side_indices(wb)
    gb, posb, eb = _side_indices(bb)
    gidx_all = jnp.concatenate([gw, gb])
    eot_all = jnp.concatenate([ew, eb])
    pos_all = jnp.concatenate([posw, P + posb])

    W1s = jnp.zeros((2, 128, H), jnp.float32)
    W1s = W1s.at[0, :32].set(W1[:, :H]).at[1, :32].set(W1[:, H:])
    b1p = jnp.zeros((1, 128), jnp.float32).at[0, :32].set(b1)
    W2p = jnp.zeros((32, 128), jnp.float32).at[:, :32].set(W2)

    xs = _sc_gather_features(white_features, black_features, gidx_all)
    y = _tc_expert_matmul(eot_all, xs, W_ft, W1s)
    aligned = _sc_gather_hidden(y, pos_all)
    out = _tc_mlp(
        aligned,
        b1p, W2p, b2.reshape(1, 32),
        W3, b3.reshape(1, 1),
        stm.reshape(B, 1),
    )
    return out
